# baseline (speedup 1.0000x reference)
import functools

import jax
import jax.numpy as jnp
from jax import lax
from jax.experimental import pallas as pl
from jax.experimental.pallas import tpu as pltpu
from jax.experimental.pallas import tpu_sc as plsc

B, L, D = 4, 8192, 1024

_info = plsc.get_sparse_core_info()
NC, NS, NL = _info.num_cores, _info.num_subcores, _info.num_lanes
NW = NC * NS
L_PER_W = L // NW
R = 8
NSLAB = L_PER_W // R
NB = 3

_mesh = plsc.VectorSubcoreMesh(core_axis_name="c", subcore_axis_name="s")

_scratch = (
    [pltpu.VMEM((B, R, D), jnp.float32) for _ in range(NB)]
    + [pltpu.VMEM((R, D), jnp.float32) for _ in range(NB)]
    + [pltpu.SemaphoreType.DMA for _ in range(2 * NB)]
)


@functools.partial(
    pl.kernel,
    mesh=_mesh,
    out_type=jax.ShapeDtypeStruct((B, L, D), jnp.float32),
    scratch_types=_scratch,
)
def _sc_add(x_hbm, emb_hbm, out_hbm, *scr):
    x_v = [scr[k] for k in range(NB)]
    emb_v = [scr[NB + k] for k in range(NB)]
    sem_in = [scr[2 * NB + k] for k in range(NB)]
    sem_out = [scr[3 * NB + k] for k in range(NB)]

    wid = lax.axis_index("s") * NC + lax.axis_index("c")
    base = wid * L_PER_W

    def issue_in(t, k):
        l0 = base + t * R
        pltpu.async_copy(emb_hbm.at[pl.ds(l0, R), :], emb_v[k], sem_in[k])
        pltpu.async_copy(x_hbm.at[:, pl.ds(l0, R), :], x_v[k], sem_in[k])

    def wait_in(k):
        pltpu.make_async_copy(
            emb_hbm.at[pl.ds(0, R), :], emb_v[k], sem_in[k]
        ).wait()
        pltpu.make_async_copy(
            x_hbm.at[:, pl.ds(0, R), :], x_v[k], sem_in[k]
        ).wait()

    def issue_out(t, k):
        l0 = base + t * R
        pltpu.async_copy(x_v[k], out_hbm.at[:, pl.ds(l0, R), :], sem_out[k])

    def wait_out(k):
        pltpu.make_async_copy(
            x_v[k], out_hbm.at[:, pl.ds(0, R), :], sem_out[k]
        ).wait()

    UNROLL = 32

    def compute(k):
        def row(r, c):
            def grp(j, c2):
                for u in range(UNROLL):
                    sl = pl.ds(j * (UNROLL * NL) + u * NL, NL)
                    e = emb_v[k][r, sl]
                    for b in range(B):
                        x_v[k][b, r, sl] = x_v[k][b, r, sl] + e
                return c2

            return lax.fori_loop(0, D // (UNROLL * NL), grp, c)

        lax.fori_loop(0, R, row, 0)

    issue_in(0, 0)
    issue_in(1, 1)

    wait_in(0)
    compute(0)
    issue_out(0, 0)
    issue_in(2, 2)

    def chunk(i, c):
        t = 1 + i * NB
        for p in range(NB):
            k = (1 + p) % NB
            wait_in(k)
            compute(k)
            issue_out(t + p, k)
            wait_out((k + NB - 1) % NB)
            @pl.when(t + p + 2 < NSLAB)
            def _(t=t, p=p, k=k):
                issue_in(t + p + 2, (k + 2) % NB)
        return c

    lax.fori_loop(0, (NSLAB - 2) // NB, chunk, 0)

    t = NSLAB - 1
    k = t % NB
    wait_in(k)
    compute(k)
    issue_out(t, k)
    wait_out((k + NB - 1) % NB)
    wait_out(k)


def kernel(x, emb):
    return _sc_add(x, emb)


# speedup vs baseline: 1.4254x; 1.4254x over previous
import functools

import jax
import jax.numpy as jnp
from jax import lax
from jax.experimental import pallas as pl
from jax.experimental.pallas import tpu as pltpu
from jax.experimental.pallas import tpu_sc as plsc

B, L, D = 4, 8192, 1024

_info = plsc.get_sparse_core_info()
NC, NS, NL = _info.num_cores, _info.num_subcores, _info.num_lanes
NW = NC * NS
L_PER_W = L // NW
R = 8
NSLAB = L_PER_W // R
NB = 3

_mesh = plsc.VectorSubcoreMesh(core_axis_name="c", subcore_axis_name="s")

_scratch = (
    [pltpu.VMEM((B, R, D), jnp.float32) for _ in range(NB)]
    + [pltpu.VMEM((R, D), jnp.float32) for _ in range(NB)]
    + [pltpu.SemaphoreType.DMA for _ in range(2 * NB)]
)


@functools.partial(
    pl.kernel,
    mesh=_mesh,
    out_type=jax.ShapeDtypeStruct((B, L, D), jnp.float32),
    scratch_types=_scratch,
)
def _sc_add(x_hbm, emb_hbm, out_hbm, *scr):
    x_v = [scr[k] for k in range(NB)]
    emb_v = [scr[NB + k] for k in range(NB)]
    sem_in = [scr[2 * NB + k] for k in range(NB)]
    sem_out = [scr[3 * NB + k] for k in range(NB)]

    wid = lax.axis_index("s") * NC + lax.axis_index("c")
    base = wid * L_PER_W

    def issue_in(t, k):
        l0 = base + t * R
        pltpu.async_copy(emb_hbm.at[pl.ds(l0, R), :], emb_v[k], sem_in[k])
        pltpu.async_copy(x_hbm.at[:, pl.ds(l0, R), :], x_v[k], sem_in[k])

    def wait_in(k):
        pltpu.make_async_copy(
            emb_hbm.at[pl.ds(0, R), :], emb_v[k], sem_in[k]
        ).wait()
        pltpu.make_async_copy(
            x_hbm.at[:, pl.ds(0, R), :], x_v[k], sem_in[k]
        ).wait()

    def issue_out(t, k):
        l0 = base + t * R
        pltpu.async_copy(x_v[k], out_hbm.at[:, pl.ds(l0, R), :], sem_out[k])

    def wait_out(k):
        pltpu.make_async_copy(
            x_v[k], out_hbm.at[:, pl.ds(0, R), :], sem_out[k]
        ).wait()

    def compute(k):
        def row(r, c):
            for j in range(D // NL):
                sl = pl.ds(j * NL, NL)
                e = emb_v[k][r, sl]
                for b in range(B):
                    x_v[k][b, r, sl] = x_v[k][b, r, sl] + e
            return c

        lax.fori_loop(0, R, row, 0)

    issue_in(0, 0)
    issue_in(1, 1)

    wait_in(0)
    compute(0)
    issue_out(0, 0)
    issue_in(2, 2)

    def chunk(i, c):
        t = 1 + i * NB
        for p in range(NB):
            k = (1 + p) % NB
            wait_in(k)
            compute(k)
            issue_out(t + p, k)
            wait_out((k + NB - 1) % NB)
            @pl.when(t + p + 2 < NSLAB)
            def _(t=t, p=p, k=k):
                issue_in(t + p + 2, (k + 2) % NB)
        return c

    lax.fori_loop(0, (NSLAB - 2) // NB, chunk, 0)

    t = NSLAB - 1
    k = t % NB
    wait_in(k)
    compute(k)
    issue_out(t, k)
    wait_out((k + NB - 1) % NB)
    wait_out(k)


def kernel(x, emb):
    return _sc_add(x, emb)
